# baseline (speedup 1.0000x reference)
import functools

import jax
import jax.numpy as jnp
from jax import lax
from jax.experimental import pallas as pl
from jax.experimental.pallas import tpu as pltpu
from jax.experimental.pallas import tpu_sc as plsc

N = 10000
E = 160000
D = 256
STEP = 0.1

NC = 2
NS = 16
NW = NC * NS

EPT = E // NW
EPT_PAD = 5008
N_PAD = 10016

CH = 64
NCH = 80
EPT2 = NCH * CH
N2 = N + 16
RPT2 = N2 // NS
DCH = 64

NB = 5
BN = N // NB

_sc_mesh = plsc.VectorSubcoreMesh(
    core_axis_name="c", subcore_axis_name="s", num_cores=NC, num_subcores=NS
)
_sc_params = pltpu.CompilerParams(
    needs_layout_passes=False, use_tc_tiling_on_sc=False
)


@functools.partial(
    pl.kernel,
    out_type=jax.ShapeDtypeStruct((NB * NW * BN,), jnp.int32),
    mesh=_sc_mesh,
    scratch_types=[
        pltpu.VMEM((EPT_PAD,), jnp.int32),
        pltpu.VMEM((N_PAD,), jnp.int32),
    ],
    compiler_params=_sc_params,
)
def _sc_deg(ei_hbm, deg_out, idx_v, deg_v):
    c = lax.axis_index("c")
    s = lax.axis_index("s")
    wid = c * NS + s

    idx_v[pl.ds(EPT_PAD - 16, 16)] = N + lax.iota(jnp.int32, 16)
    pltpu.sync_copy(ei_hbm.at[0].at[pl.ds(wid * EPT, EPT)],
                    idx_v.at[pl.ds(0, EPT)])

    zeros16 = jnp.zeros((16,), jnp.int32)

    def zbody(i, _):
        deg_v[pl.ds(i * 16, 16)] = zeros16
        return 0

    lax.fori_loop(0, N_PAD // 16, zbody, 0, unroll=4)

    def ebody(i, _):
        vals = idx_v[pl.ds(i * 16, 16)]
        cnt, last = plsc.scan_count(vals)
        plsc.addupdate_scatter(deg_v, [vals], cnt, mask=last)
        return 0

    lax.fori_loop(0, EPT_PAD // 16, ebody, 0, unroll=4)
    for b in range(NB):
        pltpu.sync_copy(deg_v.at[pl.ds(b * BN, BN)],
                        deg_out.at[pl.ds(b * (NW * BN) + wid * BN, BN)])


@functools.partial(
    pl.kernel,
    out_type=jax.ShapeDtypeStruct((NC, N2, 1, D // 2), jnp.int32),
    mesh=_sc_mesh,
    scratch_types=[
        pltpu.VMEM_SHARED((N2, 2, D // 2), jnp.bfloat16),
        pltpu.VMEM((EPT2,), jnp.int32),
        pltpu.VMEM((EPT2,), jnp.int32),
        pltpu.VMEM((CH, 2, D // 2), jnp.bfloat16),
        pltpu.VMEM((CH, 2, D // 2), jnp.bfloat16),
        pltpu.VMEM((CH, 2, D // 2), jnp.bfloat16),
        pltpu.VMEM((DCH, 1, D // 2), jnp.int32),
        pltpu.SemaphoreType.DMA,
        pltpu.SemaphoreType.DMA,
        pltpu.SemaphoreType.DMA,
        pltpu.SemaphoreType.DMA,
        pltpu.SemaphoreType.DMA,
        pltpu.SemaphoreType.DMA,
    ],
    compiler_params=_sc_params,
)
def _sc_spmm(ei_hbm, y_hbm, agg_out,
             agg_sh, idxs_v, idxd_v, b0, b1, b2, dbuf,
             g0, g1, g2, s0, s1, s2):
    bufs = (b0, b1, b2)
    gsem = (g0, g1, g2)
    ssem = (s0, s1, s2)
    c = lax.axis_index("c")
    s = lax.axis_index("s")
    wid = c * NS + s

    for k in range(8):
        idxs_v[pl.ds(EPT - 8 + 16 * k, 16)] = 16 * k + lax.iota(jnp.int32, 16)
        idxd_v[pl.ds(EPT - 8 + 16 * k, 16)] = N + lax.iota(jnp.int32, 16)
    pltpu.sync_copy(ei_hbm.at[0].at[pl.ds(wid * EPT, EPT)],
                    idxs_v.at[pl.ds(0, EPT)])
    pltpu.sync_copy(ei_hbm.at[1].at[pl.ds(wid * EPT, EPT)],
                    idxd_v.at[pl.ds(0, EPT)])

    zb = jnp.zeros((32,), jnp.bfloat16)

    def zrow(r, _):
        for j2 in range(2):
            for k in range(4):
                b0[r, j2, pl.ds(32 * k, 32)] = zb
        return 0

    lax.fori_loop(0, CH, zrow, 0)
    base_row = s * RPT2
    for t in range(RPT2 // CH):
        pltpu.sync_copy(b0.at[pl.ds(0, CH)],
                        agg_sh.at[pl.ds(base_row + t * CH, CH)])
    pltpu.sync_copy(b0.at[pl.ds(0, RPT2 % CH)],
                    agg_sh.at[pl.ds(base_row + (RPT2 // CH) * CH, RPT2 % CH)])
    plsc.subcore_barrier()

    def fire_gather(j, slot):
        pltpu.async_copy(y_hbm.at[idxs_v.at[pl.ds(j * CH, CH)]],
                         bufs[slot], gsem[slot])

    def wait_gather(slot):
        pltpu.make_async_copy(y_hbm.at[idxs_v.at[pl.ds(0, CH)]],
                              bufs[slot], gsem[slot]).wait()

    def fire_scatter(j, slot):
        pltpu.async_copy(bufs[slot], agg_sh.at[idxd_v.at[pl.ds(j * CH, CH)]],
                         ssem[slot], add=True)

    def wait_scatter(slot):
        pltpu.make_async_copy(bufs[slot], agg_sh.at[idxd_v.at[pl.ds(0, CH)]],
                              ssem[slot]).wait()

    fire_gather(0, 0)
    fire_gather(1, 1)

    def body(i, _):
        j0 = 3 * i
        for k in range(3):
            j = j0 + k
            wait_gather(k)
            fire_scatter(j, k)

            @pl.when(j >= 1)
            def _():
                wait_scatter((k + 2) % 3)

            @pl.when(j + 2 < NCH)
            def _():
                fire_gather(j + 2, (k + 2) % 3)

        return 0

    lax.fori_loop(0, NCH // 3, body, 0)
    for j in (NCH - 2, NCH - 1):
        k = j % 3
        wait_gather(k)
        fire_scatter(j, k)
        wait_scatter((k + 2) % 3)
    wait_scatter((NCH - 1) % 3)
    plsc.subcore_barrier()

    for t in range((RPT2 + DCH - 1) // DCH):
        rows = min(DCH, RPT2 - t * DCH)
        pltpu.sync_copy(agg_sh.at[pl.ds(base_row + t * DCH, rows)],
                        b0.at[pl.ds(0, rows)])

        def drow(r, _):
            for j2 in range(2):
                for k in range(4):
                    v = b0[r, j2, pl.ds(32 * k, 32)]
                    dbuf[r, 0, pl.ds(j2 * 64 + 16 * k, 16)] = plsc.bitcast(
                        v, jnp.int32)
            return 0

        lax.fori_loop(0, rows, drow, 0)
        pltpu.sync_copy(dbuf.at[pl.ds(0, rows)],
                        agg_out.at[c].at[pl.ds(base_row + t * DCH, rows)])


def _tc_pre_body(x_ref, degp_ref, y_ref):
    deg = jnp.sum(degp_ref[0], axis=0).astype(jnp.float32)
    dinv = jnp.where(deg > 0, lax.rsqrt(deg), 0.0)
    y_ref[...] = (x_ref[...] * dinv[:, None]).astype(jnp.bfloat16)


_tc_pre = pl.pallas_call(
    _tc_pre_body,
    grid=(NB,),
    in_specs=[
        pl.BlockSpec((BN, D), lambda i: (i, 0)),
        pl.BlockSpec((1, NW, BN), lambda i: (i, 0, 0)),
    ],
    out_specs=pl.BlockSpec((BN, D), lambda i: (i, 0)),
    out_shape=jax.ShapeDtypeStruct((N, D), jnp.bfloat16),
)


def _tc_base_body(x_ref, x0_ref, wt_ref, c0_ref, base_ref):
    base_ref[...] = x_ref[...] * c0_ref[...] - jnp.dot(
        x0_ref[...], wt_ref[...], preferred_element_type=jnp.float32
    )


_tc_base = pl.pallas_call(
    _tc_base_body,
    grid=(NB,),
    in_specs=[
        pl.BlockSpec((BN, D), lambda i: (i, 0)),
        pl.BlockSpec((BN, D), lambda i: (i, 0)),
        pl.BlockSpec((D, D), lambda i: (0, 0)),
        pl.BlockSpec((1, D), lambda i: (0, 0)),
    ],
    out_specs=pl.BlockSpec((BN, D), lambda i: (i, 0)),
    out_shape=jax.ShapeDtypeStruct((N, D), jnp.float32),
)


def _tc_post_body(agg_ref, degp_ref, base_ref, wse_ref, wso_ref, out_ref):
    a = agg_ref[...]
    w0 = a[0, :, 0, :]
    w1 = a[1, :, 0, :]
    ev = (lax.bitcast_convert_type(w0 << 16, jnp.float32)
          + lax.bitcast_convert_type(w1 << 16, jnp.float32))
    m = jnp.int32(-65536)
    od = (lax.bitcast_convert_type(w0 & m, jnp.float32)
          + lax.bitcast_convert_type(w1 & m, jnp.float32))
    deg = jnp.sum(degp_ref[0], axis=0).astype(jnp.float32)
    dinv = jnp.where(deg > 0, lax.rsqrt(deg), 0.0)
    out_ref[...] = base_ref[...] + jnp.dot(
        ev * dinv[:, None], wse_ref[...], preferred_element_type=jnp.float32
    ) + jnp.dot(
        od * dinv[:, None], wso_ref[...], preferred_element_type=jnp.float32
    )


_tc_post = pl.pallas_call(
    _tc_post_body,
    grid=(NB,),
    in_specs=[
        pl.BlockSpec((NC, BN, 1, D // 2), lambda i: (0, i, 0, 0)),
        pl.BlockSpec((1, NW, BN), lambda i: (i, 0, 0)),
        pl.BlockSpec((BN, D), lambda i: (i, 0)),
        pl.BlockSpec((D // 2, D), lambda i: (0, 0)),
        pl.BlockSpec((D // 2, D), lambda i: (0, 0)),
    ],
    out_specs=pl.BlockSpec((BN, D), lambda i: (i, 0)),
    out_shape=jax.ShapeDtypeStruct((N, D), jnp.float32),
)


def kernel(x, x0, edge_index, Omega, W, W_tilde):
    degp = _sc_deg(edge_index).reshape(NB, NW, BN)

    y = _tc_pre(x, degp)

    agg_i32 = _sc_spmm(edge_index, y.reshape(N, 2, D // 2))

    c0 = (1.0 - STEP * Omega).astype(jnp.float32).reshape(1, D)
    wt_s = (STEP * W_tilde).astype(jnp.float32)
    base = _tc_base(x, x0, wt_s, c0)
    ws = (STEP * (W + W.T)).astype(jnp.float32)
    out = _tc_post(agg_i32, degp, base, ws[0::2], ws[1::2])
    return out


# speedup vs baseline: 1.0106x; 1.0106x over previous
import functools

import jax
import jax.numpy as jnp
from jax import lax
from jax.experimental import pallas as pl
from jax.experimental.pallas import tpu as pltpu
from jax.experimental.pallas import tpu_sc as plsc

N = 10000
E = 160000
D = 256
STEP = 0.1

NC = 2
NS = 16
NW = NC * NS

EPT = E // NW
EPT_PAD = 5008
N_PAD = 10016

CH = 64
NCH = 80
EPT2 = NCH * CH
N2 = N + 16
RPT2 = N2 // NS
DCH = 64

NB = 5
BN = N // NB

_sc_mesh = plsc.VectorSubcoreMesh(
    core_axis_name="c", subcore_axis_name="s", num_cores=NC, num_subcores=NS
)
_sc_params = pltpu.CompilerParams(
    needs_layout_passes=False, use_tc_tiling_on_sc=False
)


@functools.partial(
    pl.kernel,
    out_type=jax.ShapeDtypeStruct((NB * NW * BN,), jnp.int32),
    mesh=_sc_mesh,
    scratch_types=[
        pltpu.VMEM((EPT_PAD,), jnp.int32),
        pltpu.VMEM((N_PAD,), jnp.int32),
    ],
    compiler_params=_sc_params,
)
def _sc_deg(ei_hbm, deg_out, idx_v, deg_v):
    c = lax.axis_index("c")
    s = lax.axis_index("s")
    wid = c * NS + s

    idx_v[pl.ds(EPT_PAD - 16, 16)] = N + lax.iota(jnp.int32, 16)
    pltpu.sync_copy(ei_hbm.at[0].at[pl.ds(wid * EPT, EPT)],
                    idx_v.at[pl.ds(0, EPT)])

    zeros16 = jnp.zeros((16,), jnp.int32)

    def zbody(i, _):
        deg_v[pl.ds(i * 16, 16)] = zeros16
        return 0

    lax.fori_loop(0, N_PAD // 16, zbody, 0, unroll=4)

    def ebody(i, _):
        vals = idx_v[pl.ds(i * 16, 16)]
        cnt, last = plsc.scan_count(vals)
        plsc.addupdate_scatter(deg_v, [vals], cnt, mask=last)
        return 0

    lax.fori_loop(0, EPT_PAD // 16, ebody, 0, unroll=4)
    for b in range(NB):
        pltpu.sync_copy(deg_v.at[pl.ds(b * BN, BN)],
                        deg_out.at[pl.ds(b * (NW * BN) + wid * BN, BN)])


@functools.partial(
    pl.kernel,
    out_type=jax.ShapeDtypeStruct((NC, N2, 1, D // 2), jnp.int32),
    mesh=_sc_mesh,
    scratch_types=[
        pltpu.VMEM_SHARED((N2, 2, D // 2), jnp.bfloat16),
        pltpu.VMEM((EPT2,), jnp.int32),
        pltpu.VMEM((EPT2,), jnp.int32),
        pltpu.VMEM((CH, 2, D // 2), jnp.bfloat16),
        pltpu.VMEM((CH, 2, D // 2), jnp.bfloat16),
        pltpu.VMEM((CH, 2, D // 2), jnp.bfloat16),
        pltpu.VMEM((DCH, 1, D // 2), jnp.int32),
        pltpu.SemaphoreType.DMA,
        pltpu.SemaphoreType.DMA,
        pltpu.SemaphoreType.DMA,
        pltpu.SemaphoreType.DMA,
        pltpu.SemaphoreType.DMA,
        pltpu.SemaphoreType.DMA,
    ],
    compiler_params=_sc_params,
)
def _sc_spmm(ei_hbm, y_hbm, agg_out,
             agg_sh, idxs_v, idxd_v, b0, b1, b2, dbuf,
             g0, g1, g2, s0, s1, s2):
    bufs = (b0, b1, b2)
    gsem = (g0, g1, g2)
    ssem = (s0, s1, s2)
    c = lax.axis_index("c")
    s = lax.axis_index("s")
    wid = c * NS + s

    for k in range(8):
        idxs_v[pl.ds(EPT - 8 + 16 * k, 16)] = 16 * k + lax.iota(jnp.int32, 16)
        idxd_v[pl.ds(EPT - 8 + 16 * k, 16)] = N + lax.iota(jnp.int32, 16)
    pltpu.sync_copy(ei_hbm.at[0].at[pl.ds(wid * EPT, EPT)],
                    idxs_v.at[pl.ds(0, EPT)])
    pltpu.sync_copy(ei_hbm.at[1].at[pl.ds(wid * EPT, EPT)],
                    idxd_v.at[pl.ds(0, EPT)])

    zb = jnp.zeros((32,), jnp.bfloat16)

    def zrow(r, _):
        for j2 in range(2):
            for k in range(4):
                b0[r, j2, pl.ds(32 * k, 32)] = zb
        return 0

    lax.fori_loop(0, CH, zrow, 0)
    base_row = s * RPT2
    for t in range(RPT2 // CH):
        pltpu.sync_copy(b0.at[pl.ds(0, CH)],
                        agg_sh.at[pl.ds(base_row + t * CH, CH)])
    pltpu.sync_copy(b0.at[pl.ds(0, RPT2 % CH)],
                    agg_sh.at[pl.ds(base_row + (RPT2 // CH) * CH, RPT2 % CH)])
    plsc.subcore_barrier()

    def fire_gather(j, slot):
        pltpu.async_copy(y_hbm.at[idxs_v.at[pl.ds(j * CH, CH)]],
                         bufs[slot], gsem[slot])

    def wait_gather(slot):
        pltpu.make_async_copy(y_hbm.at[idxs_v.at[pl.ds(0, CH)]],
                              bufs[slot], gsem[slot]).wait()

    def fire_scatter(j, slot):
        pltpu.async_copy(bufs[slot], agg_sh.at[idxd_v.at[pl.ds(j * CH, CH)]],
                         ssem[slot], add=True)

    def wait_scatter(slot):
        pltpu.make_async_copy(bufs[slot], agg_sh.at[idxd_v.at[pl.ds(0, CH)]],
                              ssem[slot]).wait()

    fire_gather(0, 0)
    fire_gather(1, 1)

    def body(i, _):
        j0 = 3 * i
        for k in range(3):
            j = j0 + k
            wait_gather(k)
            fire_scatter(j, k)

            @pl.when(j >= 1)
            def _():
                wait_scatter((k + 2) % 3)

            @pl.when(j + 2 < NCH)
            def _():
                fire_gather(j + 2, (k + 2) % 3)

        return 0

    lax.fori_loop(0, NCH // 3, body, 0)
    for j in (NCH - 2, NCH - 1):
        k = j % 3
        wait_gather(k)
        fire_scatter(j, k)
        wait_scatter((k + 2) % 3)
    wait_scatter((NCH - 1) % 3)
    plsc.subcore_barrier()

    for t in range((RPT2 + DCH - 1) // DCH):
        rows = min(DCH, RPT2 - t * DCH)
        pltpu.sync_copy(agg_sh.at[pl.ds(base_row + t * DCH, rows)],
                        b0.at[pl.ds(0, rows)])

        def drow(r, _):
            for j2 in range(2):
                for k in range(4):
                    v = b0[r, j2, pl.ds(32 * k, 32)]
                    dbuf[r, 0, pl.ds(j2 * 64 + 16 * k, 16)] = plsc.bitcast(
                        v, jnp.int32)
            return 0

        lax.fori_loop(0, rows, drow, 0)
        pltpu.sync_copy(dbuf.at[pl.ds(0, rows)],
                        agg_out.at[c].at[pl.ds(base_row + t * DCH, rows)])


def _tc_pre_body(x_ref, degp_ref, y_ref):
    deg = jnp.sum(degp_ref[0], axis=0).astype(jnp.float32)
    dinv = jnp.where(deg > 0, lax.rsqrt(deg), 0.0)
    y_ref[...] = (x_ref[...] * dinv[:, None]).astype(jnp.bfloat16)


_tc_pre = pl.pallas_call(
    _tc_pre_body,
    grid=(NB,),
    in_specs=[
        pl.BlockSpec((BN, D), lambda i: (i, 0)),
        pl.BlockSpec((1, NW, BN), lambda i: (i, 0, 0)),
    ],
    out_specs=pl.BlockSpec((BN, D), lambda i: (i, 0)),
    out_shape=jax.ShapeDtypeStruct((N, D), jnp.bfloat16),
)


def _tc_post_body(agg_ref, degp_ref, x_ref, x0_ref, wt_ref, c0_ref,
                  wse_ref, wso_ref, out_ref):
    a = agg_ref[...]
    w0 = a[0, :, 0, :]
    w1 = a[1, :, 0, :]
    ev = (lax.bitcast_convert_type(w0 << 16, jnp.float32)
          + lax.bitcast_convert_type(w1 << 16, jnp.float32))
    m = jnp.int32(-65536)
    od = (lax.bitcast_convert_type(w0 & m, jnp.float32)
          + lax.bitcast_convert_type(w1 & m, jnp.float32))
    deg = jnp.sum(degp_ref[0], axis=0).astype(jnp.float32)
    dinv = jnp.where(deg > 0, lax.rsqrt(deg), 0.0)
    base = x_ref[...] * c0_ref[...] - jnp.dot(
        x0_ref[...], wt_ref[...], preferred_element_type=jnp.float32
    )
    out_ref[...] = base + jnp.dot(
        ev * dinv[:, None], wse_ref[...], preferred_element_type=jnp.float32
    ) + jnp.dot(
        od * dinv[:, None], wso_ref[...], preferred_element_type=jnp.float32
    )


_tc_post = pl.pallas_call(
    _tc_post_body,
    grid=(NB,),
    in_specs=[
        pl.BlockSpec((NC, BN, 1, D // 2), lambda i: (0, i, 0, 0)),
        pl.BlockSpec((1, NW, BN), lambda i: (i, 0, 0)),
        pl.BlockSpec((BN, D), lambda i: (i, 0)),
        pl.BlockSpec((BN, D), lambda i: (i, 0)),
        pl.BlockSpec((D, D), lambda i: (0, 0)),
        pl.BlockSpec((1, D), lambda i: (0, 0)),
        pl.BlockSpec((D // 2, D), lambda i: (0, 0)),
        pl.BlockSpec((D // 2, D), lambda i: (0, 0)),
    ],
    out_specs=pl.BlockSpec((BN, D), lambda i: (i, 0)),
    out_shape=jax.ShapeDtypeStruct((N, D), jnp.float32),
)


def kernel(x, x0, edge_index, Omega, W, W_tilde):
    degp = _sc_deg(edge_index).reshape(NB, NW, BN)

    y = _tc_pre(x, degp)

    agg_i32 = _sc_spmm(edge_index, y.reshape(N, 2, D // 2))

    c0 = (1.0 - STEP * Omega).astype(jnp.float32).reshape(1, D)
    wt_s = (STEP * W_tilde).astype(jnp.float32)
    ws = (STEP * (W + W.T)).astype(jnp.float32)
    out = _tc_post(agg_i32, degp, x, x0, wt_s, c0, ws[0::2], ws[1::2])
    return out
